# baseline (speedup 1.0000x reference)
import functools

import jax
import jax.numpy as jnp
from jax import lax
from jax.experimental import pallas as pl
from jax.experimental.pallas import tpu as pltpu
from jax.experimental.pallas import tpu_sc as plsc

N_NODES = 10000
N_EDGES = 320000
D = 128

NC = 2
NS = 16
NW = NC * NS

N_PAD = 10240
ROWS_PER_TILE = N_PAD // NS

CHUNK = 128
NCHUNK = 82
E_PER_TILE = NCHUNK * CHUNK
E_PAD = NW * E_PER_TILE
EXTRA_PER_TILE = E_PER_TILE - N_EDGES // NW

_mesh = plsc.VectorSubcoreMesh(core_axis_name="c", subcore_axis_name="s")
_sc_params = pltpu.CompilerParams(use_tc_tiling_on_sc=False)


@functools.partial(
    pl.kernel,
    mesh=_mesh,
    compiler_params=_sc_params,
    out_type=jax.ShapeDtypeStruct((NC, N_PAD), jnp.float32),
    scratch_types=[
        pltpu.VMEM((NCHUNK, CHUNK), jnp.int32),
        pltpu.VMEM((CHUNK,), jnp.float32),
        pltpu.VMEM((ROWS_PER_TILE,), jnp.float32),
        pltpu.VMEM_SHARED((N_PAD,), jnp.float32),
    ],
)
def _deg_kernel(dst_hbm, deg_out, dst_v, ones_v, zeros_v, acc):
    cid = lax.axis_index("c")
    sid = lax.axis_index("s")
    wid = cid * NS + sid

    for i in range(CHUNK // 16):
        ones_v[pl.ds(i * 16, 16)] = jnp.ones((16,), jnp.float32)
    for i in range(ROWS_PER_TILE // 16):
        zeros_v[pl.ds(i * 16, 16)] = jnp.zeros((16,), jnp.float32)

    pltpu.sync_copy(zeros_v, acc.at[pl.ds(sid * ROWS_PER_TILE, ROWS_PER_TILE)])
    pltpu.sync_copy(dst_hbm.at[pl.ds(wid * NCHUNK, NCHUNK)], dst_v)
    plsc.subcore_barrier()

    def body(j, carry):
        pltpu.sync_copy(ones_v, acc.at[dst_v.at[j]], add=True)
        return carry

    lax.fori_loop(0, NCHUNK, body, 0)
    plsc.subcore_barrier()

    sl = pl.ds(sid * ROWS_PER_TILE, ROWS_PER_TILE)
    pltpu.sync_copy(acc.at[sl], deg_out.at[cid, sl])


@functools.partial(
    pl.kernel,
    mesh=_mesh,
    compiler_params=_sc_params,
    out_type=jax.ShapeDtypeStruct((NC, N_PAD, 2 * D), jnp.bfloat16),
    scratch_types=[
        pltpu.VMEM((NCHUNK, CHUNK), jnp.int32),
        pltpu.VMEM((NCHUNK, CHUNK), jnp.int32),
        pltpu.VMEM((CHUNK, D), jnp.bfloat16),
        pltpu.VMEM((CHUNK, D), jnp.bfloat16),
        pltpu.VMEM((8, D), jnp.bfloat16),
        pltpu.VMEM_SHARED((N_PAD, D), jnp.bfloat16),
        pltpu.SemaphoreType.DMA,
        pltpu.SemaphoreType.DMA,
    ],
)
def _agg_kernel(src_hbm, dst_hbm, h2_hbm, agg_out,
                src_v, dst_v, rows0, rows1, ztile, acc, sem0, sem1):
    cid = lax.axis_index("c")
    sid = lax.axis_index("s")
    wid = cid * NS + sid

    for r in range(8):
        for c in range(D // 32):
            ztile[r, pl.ds(c * 32, 32)] = jnp.zeros((32,), jnp.bfloat16)

    def zcopy(j, carry):
        pltpu.sync_copy(
            ztile, acc.at[pl.ds(sid * ROWS_PER_TILE + j * 8, 8)])
        return carry
    lax.fori_loop(0, ROWS_PER_TILE // 8, zcopy, 0)

    pltpu.sync_copy(src_hbm.at[pl.ds(wid * NCHUNK, NCHUNK)], src_v)
    pltpu.sync_copy(dst_hbm.at[pl.ds(wid * NCHUNK, NCHUNK)], dst_v)
    plsc.subcore_barrier()

    def gat(j, rows, sem):
        return pltpu.async_copy(h2_hbm.at[src_v.at[j]], rows, sem)

    gat(0, rows0, sem0)

    def body(g, carry):
        j0 = 2 * g
        j1 = j0 + 1
        j2 = j0 + 2
        gat(j1, rows1, sem1)
        pltpu.make_async_copy(h2_hbm.at[src_v.at[j0]], rows0, sem0).wait()
        pltpu.sync_copy(rows0, acc.at[dst_v.at[j0]], add=True)

        @pl.when(j2 < NCHUNK)
        def _():
            gat(j2, rows0, sem0)

        pltpu.make_async_copy(h2_hbm.at[src_v.at[j1]], rows1, sem1).wait()
        pltpu.sync_copy(rows1, acc.at[dst_v.at[j1]], add=True)
        return carry

    lax.fori_loop(0, NCHUNK // 2, body, 0)
    plsc.subcore_barrier()

    sl = pl.ds(sid * ROWS_PER_TILE, ROWS_PER_TILE)
    pltpu.sync_copy(acc.at[sl], agg_out.at[cid, sl, pl.ds(0, D)])


_BLK = 400
_GRID = N_NODES // _BLK


def _h2_body(x_ref, w_ref, deg_ref, h2b_ref):
    deg = jnp.maximum(deg_ref[:, 0] + deg_ref[:, 1], 1.0)
    dis = lax.rsqrt(deg)
    h = jnp.dot(x_ref[...], w_ref[...], preferred_element_type=jnp.float32)
    h2b_ref[...] = (h * dis[:, None]).astype(jnp.bfloat16)


def _h2_call(x, W, degp_t):
    return pl.pallas_call(
        _h2_body,
        grid=(_GRID,),
        in_specs=[
            pl.BlockSpec((_BLK, D), lambda i: (i, 0)),
            pl.BlockSpec((D, D), lambda i: (0, 0)),
            pl.BlockSpec((_BLK, NC), lambda i: (i, 0)),
        ],
        out_specs=pl.BlockSpec((_BLK, D), lambda i: (i, 0)),
        out_shape=jax.ShapeDtypeStruct((N_NODES, D), jnp.bfloat16),
    )(x, W, degp_t)


def _out_body(agg_ref, deg_ref, b_ref, out_ref):
    u = agg_ref[...]
    f_lo = lax.bitcast_convert_type(u << 16, jnp.float32)
    f_hi = lax.bitcast_convert_type(u & jnp.uint32(0xFFFF0000), jnp.float32)
    p_lo = f_lo[0] + f_lo[1]
    p_hi = f_hi[0] + f_hi[1]
    s = jnp.concatenate([p_lo[:, :D // 2], p_hi[:, :D // 2]], axis=1)
    deg = jnp.maximum(deg_ref[:, 0] + deg_ref[:, 1], 1.0)
    dis = lax.rsqrt(deg)
    out_ref[...] = jnp.maximum(s * dis[:, None] + b_ref[...], 0.0)


_OBLK = 400
_OGRID = N_NODES // _OBLK


def _out_call(agg_u32, degp_t, b2):
    return pl.pallas_call(
        _out_body,
        grid=(_OGRID,),
        in_specs=[
            pl.BlockSpec((NC, _OBLK, D), lambda i: (0, i, 0)),
            pl.BlockSpec((_OBLK, NC), lambda i: (i, 0)),
            pl.BlockSpec((1, D), lambda i: (0, 0)),
        ],
        out_specs=pl.BlockSpec((_OBLK, D), lambda i: (i, 0)),
        out_shape=jax.ShapeDtypeStruct((N_NODES, D), jnp.float32),
    )(agg_u32, degp_t, b2)


def kernel(x, edge_index, W, b):
    src = edge_index[0].astype(jnp.int32)
    dst = edge_index[1].astype(jnp.int32)

    n_fill = E_PAD - N_EDGES - N_NODES
    loop = jnp.arange(N_NODES, dtype=jnp.int32)
    karr = jnp.arange(n_fill, dtype=jnp.int32)
    ex_src = jnp.concatenate([loop, karr % N_NODES])
    ex_dst = jnp.concatenate([loop, N_NODES + karr % (N_PAD - N_NODES)])

    def layout(real, extra):
        return jnp.concatenate(
            [real.reshape(NW, N_EDGES // NW), extra.reshape(NW, EXTRA_PER_TILE)],
            axis=1).reshape(NW * NCHUNK, CHUNK)

    src_all = layout(src, ex_src)
    dst_all = layout(dst, ex_dst)

    degp = _deg_kernel(dst_all)
    degp_t = degp.T[:N_NODES]

    perm = (jnp.arange(D) // 2) + (D // 2) * (jnp.arange(D) % 2)
    h2b = _h2_call(x, W[:, perm], degp_t)

    agg = _agg_kernel(src_all, dst_all, h2b)

    agg_u32 = lax.bitcast_convert_type(
        agg.reshape(NC, N_PAD, D, 2), jnp.uint32)

    return _out_call(agg_u32, degp_t, b.reshape(1, D))


# speedup vs baseline: 1.4394x; 1.4394x over previous
import functools

import jax
import jax.numpy as jnp
from jax import lax
from jax.experimental import pallas as pl
from jax.experimental.pallas import tpu as pltpu
from jax.experimental.pallas import tpu_sc as plsc

N_NODES = 10000
N_EDGES = 320000
D = 128

NC = 2
NS = 16
NW = NC * NS

N_PAD = 10240
ROWS_PER_TILE = N_PAD // NS

CHUNK = 128
NCHUNK = 82
E_PER_TILE = NCHUNK * CHUNK
E_PAD = NW * E_PER_TILE
EXTRA_PER_TILE = E_PER_TILE - N_EDGES // NW

_mesh = plsc.VectorSubcoreMesh(core_axis_name="c", subcore_axis_name="s")
_sc_params = pltpu.CompilerParams(use_tc_tiling_on_sc=False)


@functools.partial(
    pl.kernel,
    mesh=_mesh,
    compiler_params=_sc_params,
    out_type=jax.ShapeDtypeStruct((NC, N_PAD), jnp.float32),
    scratch_types=[
        pltpu.VMEM((NCHUNK, CHUNK), jnp.int32),
        pltpu.VMEM((CHUNK,), jnp.float32),
        pltpu.VMEM((ROWS_PER_TILE,), jnp.float32),
        pltpu.VMEM_SHARED((N_PAD,), jnp.float32),
    ],
)
def _deg_kernel(dst_hbm, deg_out, dst_v, ones_v, zeros_v, acc):
    cid = lax.axis_index("c")
    sid = lax.axis_index("s")
    wid = cid * NS + sid

    for i in range(CHUNK // 16):
        ones_v[pl.ds(i * 16, 16)] = jnp.ones((16,), jnp.float32)
    for i in range(ROWS_PER_TILE // 16):
        zeros_v[pl.ds(i * 16, 16)] = jnp.zeros((16,), jnp.float32)

    pltpu.sync_copy(zeros_v, acc.at[pl.ds(sid * ROWS_PER_TILE, ROWS_PER_TILE)])
    pltpu.sync_copy(dst_hbm.at[pl.ds(wid * NCHUNK, NCHUNK)], dst_v)
    plsc.subcore_barrier()

    def body(j, carry):
        pltpu.sync_copy(ones_v, acc.at[dst_v.at[j]], add=True)
        return carry

    lax.fori_loop(0, NCHUNK, body, 0)
    plsc.subcore_barrier()

    sl = pl.ds(sid * ROWS_PER_TILE, ROWS_PER_TILE)
    pltpu.sync_copy(acc.at[sl], deg_out.at[cid, sl])


@functools.partial(
    pl.kernel,
    mesh=_mesh,
    compiler_params=_sc_params,
    out_type=jax.ShapeDtypeStruct((NC, N_PAD, D), jnp.bfloat16),
    scratch_types=[
        pltpu.VMEM((NCHUNK, CHUNK), jnp.int32),
        pltpu.VMEM((NCHUNK, CHUNK), jnp.int32),
        pltpu.VMEM((CHUNK, D), jnp.bfloat16),
        pltpu.VMEM((CHUNK, D), jnp.bfloat16),
        pltpu.VMEM((8, D), jnp.bfloat16),
        pltpu.VMEM_SHARED((N_PAD, D), jnp.bfloat16),
        pltpu.SemaphoreType.DMA,
        pltpu.SemaphoreType.DMA,
    ],
)
def _agg_kernel(src_hbm, dst_hbm, h2_hbm, agg_out,
                src_v, dst_v, rows0, rows1, ztile, acc, sem0, sem1):
    cid = lax.axis_index("c")
    sid = lax.axis_index("s")
    wid = cid * NS + sid

    for r in range(8):
        for c in range(D // 32):
            ztile[r, pl.ds(c * 32, 32)] = jnp.zeros((32,), jnp.bfloat16)

    def zcopy(j, carry):
        pltpu.sync_copy(
            ztile, acc.at[pl.ds(sid * ROWS_PER_TILE + j * 8, 8)])
        return carry
    lax.fori_loop(0, ROWS_PER_TILE // 8, zcopy, 0)

    pltpu.sync_copy(src_hbm.at[pl.ds(wid * NCHUNK, NCHUNK)], src_v)
    pltpu.sync_copy(dst_hbm.at[pl.ds(wid * NCHUNK, NCHUNK)], dst_v)
    plsc.subcore_barrier()

    def gat(j, rows, sem):
        return pltpu.async_copy(h2_hbm.at[src_v.at[j]], rows, sem)

    gat(0, rows0, sem0)

    def body(g, carry):
        j0 = 2 * g
        j1 = j0 + 1
        j2 = j0 + 2
        gat(j1, rows1, sem1)
        pltpu.make_async_copy(h2_hbm.at[src_v.at[j0]], rows0, sem0).wait()
        pltpu.sync_copy(rows0, acc.at[dst_v.at[j0]], add=True)

        @pl.when(j2 < NCHUNK)
        def _():
            gat(j2, rows0, sem0)

        pltpu.make_async_copy(h2_hbm.at[src_v.at[j1]], rows1, sem1).wait()
        pltpu.sync_copy(rows1, acc.at[dst_v.at[j1]], add=True)
        return carry

    lax.fori_loop(0, NCHUNK // 2, body, 0)
    plsc.subcore_barrier()

    sl = pl.ds(sid * ROWS_PER_TILE, ROWS_PER_TILE)
    pltpu.sync_copy(acc.at[sl], agg_out.at[cid, sl])


_BLK = 400
_GRID = N_NODES // _BLK


def _h2_body(x_ref, w_ref, deg_ref, h2b_ref):
    deg = jnp.maximum(deg_ref[:, 0] + deg_ref[:, 1], 1.0)
    dis = lax.rsqrt(deg)
    h = jnp.dot(x_ref[...], w_ref[...], preferred_element_type=jnp.float32)
    h2b_ref[...] = (h * dis[:, None]).astype(jnp.bfloat16)


def _h2_call(x, W, degp_t):
    return pl.pallas_call(
        _h2_body,
        grid=(_GRID,),
        in_specs=[
            pl.BlockSpec((_BLK, D), lambda i: (i, 0)),
            pl.BlockSpec((D, D), lambda i: (0, 0)),
            pl.BlockSpec((_BLK, NC), lambda i: (i, 0)),
        ],
        out_specs=pl.BlockSpec((_BLK, D), lambda i: (i, 0)),
        out_shape=jax.ShapeDtypeStruct((N_NODES, D), jnp.bfloat16),
    )(x, W, degp_t)


def _out_body(agg_ref, deg_ref, b_ref, out_ref):
    deg = jnp.maximum(deg_ref[:, 0] + deg_ref[:, 1], 1.0)
    dis = lax.rsqrt(deg)
    s = agg_ref[0].astype(jnp.float32) + agg_ref[1].astype(jnp.float32)
    out_ref[...] = jnp.maximum(s * dis[:, None] + b_ref[...], 0.0)


_OBLK = 400
_OGRID = N_NODES // _OBLK


def _out_call(agg, degp_t, b2):
    return pl.pallas_call(
        _out_body,
        grid=(_OGRID,),
        in_specs=[
            pl.BlockSpec((NC, _OBLK, D), lambda i: (0, i, 0)),
            pl.BlockSpec((_OBLK, NC), lambda i: (i, 0)),
            pl.BlockSpec((1, D), lambda i: (0, 0)),
        ],
        out_specs=pl.BlockSpec((_OBLK, D), lambda i: (i, 0)),
        out_shape=jax.ShapeDtypeStruct((N_NODES, D), jnp.float32),
    )(agg, degp_t, b2)


def kernel(x, edge_index, W, b):
    src = edge_index[0].astype(jnp.int32)
    dst = edge_index[1].astype(jnp.int32)

    n_fill = E_PAD - N_EDGES - N_NODES
    loop = jnp.arange(N_NODES, dtype=jnp.int32)
    karr = jnp.arange(n_fill, dtype=jnp.int32)
    ex_src = jnp.concatenate([loop, karr % N_NODES])
    ex_dst = jnp.concatenate([loop, N_NODES + karr % (N_PAD - N_NODES)])

    def layout(real, extra):
        return jnp.concatenate(
            [real.reshape(NW, N_EDGES // NW), extra.reshape(NW, EXTRA_PER_TILE)],
            axis=1).reshape(NW * NCHUNK, CHUNK)

    src_all = layout(src, ex_src)
    dst_all = layout(dst, ex_dst)

    degp = _deg_kernel(dst_all)
    degp_t = degp.T

    h2b = _h2_call(x, W, degp_t)

    agg = _agg_kernel(src_all, dst_all, h2b)

    return _out_call(agg, degp_t, b.reshape(1, D))


# speedup vs baseline: 1.4897x; 1.0349x over previous
import functools

import jax
import jax.numpy as jnp
from jax import lax
from jax.experimental import pallas as pl
from jax.experimental.pallas import tpu as pltpu
from jax.experimental.pallas import tpu_sc as plsc

N_NODES = 10000
N_EDGES = 320000
D = 128

NC = 2
NS = 16
NW = NC * NS

N_PAD = 10240
ROWS_PER_TILE = N_PAD // NS

CHUNK = 128
NCHUNK = 82
E_PER_TILE = NCHUNK * CHUNK
E_PAD = NW * E_PER_TILE
EXTRA_PER_TILE = E_PER_TILE - N_EDGES // NW

_mesh = plsc.VectorSubcoreMesh(core_axis_name="c", subcore_axis_name="s")
_sc_params = pltpu.CompilerParams(use_tc_tiling_on_sc=False)


@functools.partial(
    pl.kernel,
    mesh=_mesh,
    compiler_params=_sc_params,
    out_type=jax.ShapeDtypeStruct((NC, N_PAD), jnp.float32),
    scratch_types=[
        pltpu.VMEM((NCHUNK, CHUNK), jnp.int32),
        pltpu.VMEM((CHUNK,), jnp.float32),
        pltpu.VMEM((ROWS_PER_TILE,), jnp.float32),
        pltpu.VMEM_SHARED((N_PAD,), jnp.float32),
    ],
)
def _deg_kernel(dst_hbm, deg_out, dst_v, ones_v, zeros_v, acc):
    cid = lax.axis_index("c")
    sid = lax.axis_index("s")
    wid = cid * NS + sid

    for i in range(CHUNK // 16):
        ones_v[pl.ds(i * 16, 16)] = jnp.ones((16,), jnp.float32)
    for i in range(ROWS_PER_TILE // 16):
        zeros_v[pl.ds(i * 16, 16)] = jnp.zeros((16,), jnp.float32)

    pltpu.sync_copy(zeros_v, acc.at[pl.ds(sid * ROWS_PER_TILE, ROWS_PER_TILE)])
    pltpu.sync_copy(dst_hbm.at[pl.ds(wid * NCHUNK, NCHUNK)], dst_v)
    plsc.subcore_barrier()

    def body(j, carry):
        pltpu.sync_copy(ones_v, acc.at[dst_v.at[j]], add=True)
        return carry

    lax.fori_loop(0, NCHUNK, body, 0)
    plsc.subcore_barrier()

    sl = pl.ds(sid * ROWS_PER_TILE, ROWS_PER_TILE)
    pltpu.sync_copy(acc.at[sl], deg_out.at[cid, sl])


@functools.partial(
    pl.kernel,
    mesh=_mesh,
    compiler_params=_sc_params,
    out_type=jax.ShapeDtypeStruct((NC, N_PAD, D), jnp.bfloat16),
    scratch_types=[
        pltpu.VMEM((NCHUNK, CHUNK), jnp.int32),
        pltpu.VMEM((NCHUNK, CHUNK), jnp.int32),
        pltpu.VMEM((CHUNK, D), jnp.bfloat16),
        pltpu.VMEM((CHUNK, D), jnp.bfloat16),
        pltpu.VMEM((8, D), jnp.bfloat16),
        pltpu.VMEM_SHARED((N_PAD, D), jnp.bfloat16),
        pltpu.SemaphoreType.DMA,
        pltpu.SemaphoreType.DMA,
    ],
)
def _agg_kernel(src_hbm, dst_hbm, h2_hbm, agg_out,
                src_v, dst_v, rows0, rows1, ztile, acc, sem0, sem1):
    cid = lax.axis_index("c")
    sid = lax.axis_index("s")
    wid = cid * NS + sid

    for r in range(8):
        for c in range(D // 32):
            ztile[r, pl.ds(c * 32, 32)] = jnp.zeros((32,), jnp.bfloat16)

    def zcopy(j, carry):
        pltpu.sync_copy(
            ztile, acc.at[pl.ds(sid * ROWS_PER_TILE + j * 8, 8)])
        return carry
    lax.fori_loop(0, ROWS_PER_TILE // 8, zcopy, 0)

    pltpu.sync_copy(src_hbm.at[pl.ds(wid * NCHUNK, NCHUNK)], src_v)
    pltpu.sync_copy(dst_hbm.at[pl.ds(wid * NCHUNK, NCHUNK)], dst_v)
    plsc.subcore_barrier()

    def gat(j, rows, sem):
        return pltpu.async_copy(h2_hbm.at[src_v.at[j]], rows, sem)

    gat(0, rows0, sem0)

    def body(g, carry):
        j0 = 2 * g
        j1 = j0 + 1
        j2 = j0 + 2
        gat(j1, rows1, sem1)
        pltpu.make_async_copy(h2_hbm.at[src_v.at[j0]], rows0, sem0).wait()
        pltpu.sync_copy(rows0, acc.at[dst_v.at[j0]], add=True)

        @pl.when(j2 < NCHUNK)
        def _():
            gat(j2, rows0, sem0)

        pltpu.make_async_copy(h2_hbm.at[src_v.at[j1]], rows1, sem1).wait()
        pltpu.sync_copy(rows1, acc.at[dst_v.at[j1]], add=True)
        return carry

    lax.fori_loop(0, NCHUNK // 2, body, 0)
    plsc.subcore_barrier()

    sl = pl.ds(sid * ROWS_PER_TILE, ROWS_PER_TILE)
    pltpu.sync_copy(acc.at[sl], agg_out.at[cid, sl])


_BLK = 512
_GRID = N_PAD // _BLK


def _h2_body(x_ref, w_ref, deg_ref, h2b_ref):
    deg = jnp.maximum(deg_ref[0, :] + deg_ref[1, :], 1.0)
    dis = lax.rsqrt(deg)
    h = jnp.dot(x_ref[...], w_ref[...], preferred_element_type=jnp.float32)
    h2b_ref[...] = (h * dis[:, None]).astype(jnp.bfloat16)


def _h2_call(x_pad, W, degp):
    return pl.pallas_call(
        _h2_body,
        grid=(_GRID,),
        in_specs=[
            pl.BlockSpec((_BLK, D), lambda i: (i, 0)),
            pl.BlockSpec((D, D), lambda i: (0, 0)),
            pl.BlockSpec((NC, _BLK), lambda i: (0, i)),
        ],
        out_specs=pl.BlockSpec((_BLK, D), lambda i: (i, 0)),
        out_shape=jax.ShapeDtypeStruct((N_PAD, D), jnp.bfloat16),
    )(x_pad, W, degp)


def _out_body(agg_ref, deg_ref, b_ref, out_ref):
    deg = jnp.maximum(deg_ref[:, 0] + deg_ref[:, 1], 1.0)
    dis = lax.rsqrt(deg)
    s = agg_ref[0].astype(jnp.float32) + agg_ref[1].astype(jnp.float32)
    out_ref[...] = jnp.maximum(s * dis[:, None] + b_ref[...], 0.0)


_OBLK = 400
_OGRID = N_NODES // _OBLK


def _out_call(agg, degp_t, b2):
    return pl.pallas_call(
        _out_body,
        grid=(_OGRID,),
        in_specs=[
            pl.BlockSpec((NC, _OBLK, D), lambda i: (0, i, 0)),
            pl.BlockSpec((_OBLK, NC), lambda i: (i, 0)),
            pl.BlockSpec((1, D), lambda i: (0, 0)),
        ],
        out_specs=pl.BlockSpec((_OBLK, D), lambda i: (i, 0)),
        out_shape=jax.ShapeDtypeStruct((N_NODES, D), jnp.float32),
    )(agg, degp_t, b2)


def kernel(x, edge_index, W, b):
    src = edge_index[0].astype(jnp.int32)
    dst = edge_index[1].astype(jnp.int32)

    n_fill = E_PAD - N_EDGES - N_NODES
    loop = jnp.arange(N_NODES, dtype=jnp.int32)
    karr = jnp.arange(n_fill, dtype=jnp.int32)
    ex_src = jnp.concatenate([loop, karr % N_NODES])
    ex_dst = jnp.concatenate([loop, N_NODES + karr % (N_PAD - N_NODES)])

    def layout(real, extra):
        return jnp.concatenate(
            [real.reshape(NW, N_EDGES // NW), extra.reshape(NW, EXTRA_PER_TILE)],
            axis=1).reshape(NW * NCHUNK, CHUNK)

    src_all = layout(src, ex_src)
    dst_all = layout(dst, ex_dst)

    degp = _deg_kernel(dst_all)

    x_pad = jnp.pad(x, ((0, N_PAD - N_NODES), (0, 0)))
    h2b = _h2_call(x_pad, W, degp)

    agg = _agg_kernel(src_all, dst_all, h2b)

    return _out_call(agg, degp.T, b.reshape(1, D))


# speedup vs baseline: 1.5319x; 1.0283x over previous
import functools

import jax
import jax.numpy as jnp
from jax import lax
from jax.experimental import pallas as pl
from jax.experimental.pallas import tpu as pltpu
from jax.experimental.pallas import tpu_sc as plsc

N_NODES = 10000
N_EDGES = 320000
D = 128

NC = 2
NS = 16
NW = NC * NS

N_PAD = 10240
ROWS_PER_TILE = N_PAD // NS

CHUNK = 128
NCHUNK = 82
E_PER_TILE = NCHUNK * CHUNK
E_PAD = NW * E_PER_TILE
EXTRA_PER_TILE = E_PER_TILE - N_EDGES // NW

_mesh = plsc.VectorSubcoreMesh(core_axis_name="c", subcore_axis_name="s")
_sc_params = pltpu.CompilerParams(use_tc_tiling_on_sc=False)


@functools.partial(
    pl.kernel,
    mesh=_mesh,
    compiler_params=_sc_params,
    out_type=jax.ShapeDtypeStruct((NC, N_PAD), jnp.float32),
    scratch_types=[
        pltpu.VMEM((NCHUNK, CHUNK), jnp.int32),
        pltpu.VMEM((CHUNK,), jnp.float32),
        pltpu.VMEM((ROWS_PER_TILE,), jnp.float32),
        pltpu.VMEM_SHARED((N_PAD,), jnp.float32),
        pltpu.SemaphoreType.DMA,
    ],
)
def _deg_kernel(dst_hbm, deg_out, dst_v, ones_v, zeros_v, acc, sem):
    cid = lax.axis_index("c")
    sid = lax.axis_index("s")
    wid = cid * NS + sid

    for i in range(CHUNK // 16):
        ones_v[pl.ds(i * 16, 16)] = jnp.ones((16,), jnp.float32)
    for i in range(ROWS_PER_TILE // 16):
        zeros_v[pl.ds(i * 16, 16)] = jnp.zeros((16,), jnp.float32)

    pltpu.sync_copy(zeros_v, acc.at[pl.ds(sid * ROWS_PER_TILE, ROWS_PER_TILE)])
    pltpu.sync_copy(dst_hbm.at[pl.ds(wid * NCHUNK, NCHUNK)], dst_v)
    plsc.subcore_barrier()

    def body(j, carry):
        pltpu.async_copy(ones_v, acc.at[dst_v.at[j]], sem, add=True)
        return carry

    lax.fori_loop(0, NCHUNK, body, 0)

    def drain(j, carry):
        pltpu.make_async_copy(ones_v, acc.at[dst_v.at[j]], sem).wait()
        return carry

    lax.fori_loop(0, NCHUNK, drain, 0)
    plsc.subcore_barrier()

    sl = pl.ds(sid * ROWS_PER_TILE, ROWS_PER_TILE)
    pltpu.sync_copy(acc.at[sl], deg_out.at[cid, sl])


@functools.partial(
    pl.kernel,
    mesh=_mesh,
    compiler_params=_sc_params,
    out_type=jax.ShapeDtypeStruct((NC, N_PAD, D), jnp.bfloat16),
    scratch_types=[
        pltpu.VMEM((NCHUNK, CHUNK), jnp.int32),
        pltpu.VMEM((NCHUNK, CHUNK), jnp.int32),
        pltpu.VMEM((CHUNK, D), jnp.bfloat16),
        pltpu.VMEM((CHUNK, D), jnp.bfloat16),
        pltpu.VMEM((8, D), jnp.bfloat16),
        pltpu.VMEM_SHARED((N_PAD, D), jnp.bfloat16),
        pltpu.SemaphoreType.DMA,
        pltpu.SemaphoreType.DMA,
    ],
)
def _agg_kernel(src_hbm, dst_hbm, h2_hbm, agg_out,
                src_v, dst_v, rows0, rows1, ztile, acc, sem0, sem1):
    cid = lax.axis_index("c")
    sid = lax.axis_index("s")
    wid = cid * NS + sid

    for r in range(8):
        for c in range(D // 32):
            ztile[r, pl.ds(c * 32, 32)] = jnp.zeros((32,), jnp.bfloat16)

    def zcopy(j, carry):
        pltpu.sync_copy(
            ztile, acc.at[pl.ds(sid * ROWS_PER_TILE + j * 8, 8)])
        return carry
    lax.fori_loop(0, ROWS_PER_TILE // 8, zcopy, 0)

    pltpu.sync_copy(src_hbm.at[pl.ds(wid * NCHUNK, NCHUNK)], src_v)
    pltpu.sync_copy(dst_hbm.at[pl.ds(wid * NCHUNK, NCHUNK)], dst_v)
    plsc.subcore_barrier()

    def gat(j, rows, sem):
        return pltpu.async_copy(h2_hbm.at[src_v.at[j]], rows, sem)

    gat(0, rows0, sem0)

    def body(g, carry):
        j0 = 2 * g
        j1 = j0 + 1
        j2 = j0 + 2
        gat(j1, rows1, sem1)
        pltpu.make_async_copy(h2_hbm.at[src_v.at[j0]], rows0, sem0).wait()
        pltpu.sync_copy(rows0, acc.at[dst_v.at[j0]], add=True)

        @pl.when(j2 < NCHUNK)
        def _():
            gat(j2, rows0, sem0)

        pltpu.make_async_copy(h2_hbm.at[src_v.at[j1]], rows1, sem1).wait()
        pltpu.sync_copy(rows1, acc.at[dst_v.at[j1]], add=True)
        return carry

    lax.fori_loop(0, NCHUNK // 2, body, 0)
    plsc.subcore_barrier()

    sl = pl.ds(sid * ROWS_PER_TILE, ROWS_PER_TILE)
    pltpu.sync_copy(acc.at[sl], agg_out.at[cid, sl])


_BLK = 512
_GRID = N_PAD // _BLK


def _h2_body(x_ref, w_ref, deg_ref, h2b_ref):
    deg = jnp.maximum(deg_ref[0, :] + deg_ref[1, :], 1.0)
    dis = lax.rsqrt(deg)
    h = jnp.dot(x_ref[...], w_ref[...], preferred_element_type=jnp.float32)
    h2b_ref[...] = (h * dis[:, None]).astype(jnp.bfloat16)


def _h2_call(x_pad, W, degp):
    return pl.pallas_call(
        _h2_body,
        grid=(_GRID,),
        in_specs=[
            pl.BlockSpec((_BLK, D), lambda i: (i, 0)),
            pl.BlockSpec((D, D), lambda i: (0, 0)),
            pl.BlockSpec((NC, _BLK), lambda i: (0, i)),
        ],
        out_specs=pl.BlockSpec((_BLK, D), lambda i: (i, 0)),
        out_shape=jax.ShapeDtypeStruct((N_PAD, D), jnp.bfloat16),
    )(x_pad, W, degp)


def _out_body(agg_ref, deg_ref, b_ref, out_ref):
    deg = jnp.maximum(deg_ref[:, 0] + deg_ref[:, 1], 1.0)
    dis = lax.rsqrt(deg)
    s = agg_ref[0].astype(jnp.float32) + agg_ref[1].astype(jnp.float32)
    out_ref[...] = jnp.maximum(s * dis[:, None] + b_ref[...], 0.0)


_OBLK = 400
_OGRID = N_NODES // _OBLK


def _out_call(agg, degp_t, b2):
    return pl.pallas_call(
        _out_body,
        grid=(_OGRID,),
        in_specs=[
            pl.BlockSpec((NC, _OBLK, D), lambda i: (0, i, 0)),
            pl.BlockSpec((_OBLK, NC), lambda i: (i, 0)),
            pl.BlockSpec((1, D), lambda i: (0, 0)),
        ],
        out_specs=pl.BlockSpec((_OBLK, D), lambda i: (i, 0)),
        out_shape=jax.ShapeDtypeStruct((N_NODES, D), jnp.float32),
    )(agg, degp_t, b2)


def kernel(x, edge_index, W, b):
    src = edge_index[0].astype(jnp.int32)
    dst = edge_index[1].astype(jnp.int32)

    n_fill = E_PAD - N_EDGES - N_NODES
    loop = jnp.arange(N_NODES, dtype=jnp.int32)
    karr = jnp.arange(n_fill, dtype=jnp.int32)
    ex_src = jnp.concatenate([loop, karr % N_NODES])
    ex_dst = jnp.concatenate([loop, N_NODES + karr % (N_PAD - N_NODES)])

    def layout(real, extra):
        return jnp.concatenate(
            [real.reshape(NW, N_EDGES // NW), extra.reshape(NW, EXTRA_PER_TILE)],
            axis=1).reshape(NW * NCHUNK, CHUNK)

    src_all = layout(src, ex_src)
    dst_all = layout(dst, ex_dst)

    degp = _deg_kernel(dst_all)

    x_pad = jnp.pad(x, ((0, N_PAD - N_NODES), (0, 0)))
    h2b = _h2_call(x_pad, W, degp)

    agg = _agg_kernel(src_all, dst_all, h2b)

    return _out_call(agg, degp.T, b.reshape(1, D))


# speedup vs baseline: 1.5655x; 1.0220x over previous
import functools

import jax
import jax.numpy as jnp
from jax import lax
from jax.experimental import pallas as pl
from jax.experimental.pallas import tpu as pltpu
from jax.experimental.pallas import tpu_sc as plsc

N_NODES = 10000
N_EDGES = 320000
D = 128

NC = 2
NS = 16
NW = NC * NS

N_PAD = 10240
ROWS_PER_TILE = N_PAD // NS

CHUNK = 128
NCHUNK = 82
E_PER_TILE = NCHUNK * CHUNK
E_PAD = NW * E_PER_TILE
EXTRA_PER_TILE = E_PER_TILE - N_EDGES // NW

_mesh = plsc.VectorSubcoreMesh(core_axis_name="c", subcore_axis_name="s")
_sc_params = pltpu.CompilerParams(use_tc_tiling_on_sc=False)


@functools.partial(
    pl.kernel,
    mesh=_mesh,
    compiler_params=_sc_params,
    out_type=jax.ShapeDtypeStruct((NC, N_PAD), jnp.float32),
    scratch_types=[
        pltpu.VMEM((NCHUNK, CHUNK), jnp.int32),
        pltpu.VMEM((CHUNK,), jnp.float32),
        pltpu.VMEM((ROWS_PER_TILE,), jnp.float32),
        pltpu.VMEM_SHARED((N_PAD,), jnp.float32),
        pltpu.SemaphoreType.DMA,
    ],
)
def _deg_kernel(dst_hbm, deg_out, dst_v, ones_v, zeros_v, acc, sem):
    cid = lax.axis_index("c")
    sid = lax.axis_index("s")
    wid = cid * NS + sid

    for i in range(CHUNK // 16):
        ones_v[pl.ds(i * 16, 16)] = jnp.ones((16,), jnp.float32)
    for i in range(ROWS_PER_TILE // 16):
        zeros_v[pl.ds(i * 16, 16)] = jnp.zeros((16,), jnp.float32)

    pltpu.sync_copy(zeros_v, acc.at[pl.ds(sid * ROWS_PER_TILE, ROWS_PER_TILE)])
    pltpu.sync_copy(dst_hbm.at[pl.ds(wid * NCHUNK, NCHUNK)], dst_v)
    plsc.subcore_barrier()

    def body(j, carry):
        pltpu.async_copy(ones_v, acc.at[dst_v.at[j]], sem, add=True)
        return carry

    lax.fori_loop(0, NCHUNK, body, 0)

    def drain(j, carry):
        pltpu.make_async_copy(ones_v, acc.at[dst_v.at[j]], sem).wait()
        return carry

    lax.fori_loop(0, NCHUNK, drain, 0)
    plsc.subcore_barrier()

    sl = pl.ds(sid * ROWS_PER_TILE, ROWS_PER_TILE)
    pltpu.sync_copy(acc.at[sl], deg_out.at[cid, sl])


@functools.partial(
    pl.kernel,
    mesh=_mesh,
    compiler_params=_sc_params,
    out_type=jax.ShapeDtypeStruct((NC, N_PAD, D), jnp.bfloat16),
    scratch_types=[
        pltpu.VMEM((NCHUNK, CHUNK), jnp.int32),
        pltpu.VMEM((NCHUNK, CHUNK), jnp.int32),
        pltpu.VMEM((CHUNK, D), jnp.bfloat16),
        pltpu.VMEM((CHUNK, D), jnp.bfloat16),
        pltpu.VMEM((8, D), jnp.bfloat16),
        pltpu.VMEM_SHARED((N_PAD, D), jnp.bfloat16),
        pltpu.SemaphoreType.DMA,
        pltpu.SemaphoreType.DMA,
    ],
)
def _agg_kernel(src_hbm, dst_hbm, h2_hbm, agg_out,
                src_v, dst_v, rows0, rows1, ztile, acc, sem0, sem1):
    cid = lax.axis_index("c")
    sid = lax.axis_index("s")
    wid = cid * NS + sid

    for r in range(8):
        for c in range(D // 32):
            ztile[r, pl.ds(c * 32, 32)] = jnp.zeros((32,), jnp.bfloat16)

    def zcopy(j, carry):
        pltpu.async_copy(
            ztile, acc.at[pl.ds(sid * ROWS_PER_TILE + j * 8, 8)], sem0)
        return carry
    lax.fori_loop(0, ROWS_PER_TILE // 8, zcopy, 0)
    pltpu.async_copy(src_hbm.at[pl.ds(wid * NCHUNK, NCHUNK)], src_v, sem1)
    pltpu.async_copy(dst_hbm.at[pl.ds(wid * NCHUNK, NCHUNK)], dst_v, sem1)

    def zdrain(j, carry):
        pltpu.make_async_copy(
            ztile, acc.at[pl.ds(sid * ROWS_PER_TILE + j * 8, 8)], sem0).wait()
        return carry
    lax.fori_loop(0, ROWS_PER_TILE // 8, zdrain, 0)
    pltpu.make_async_copy(src_hbm.at[pl.ds(wid * NCHUNK, NCHUNK)], src_v,
                          sem1).wait()
    pltpu.make_async_copy(dst_hbm.at[pl.ds(wid * NCHUNK, NCHUNK)], dst_v,
                          sem1).wait()
    plsc.subcore_barrier()

    def gat(j, rows, sem):
        return pltpu.async_copy(h2_hbm.at[src_v.at[j]], rows, sem)

    gat(0, rows0, sem0)

    def body(g, carry):
        j0 = 2 * g
        j1 = j0 + 1
        j2 = j0 + 2
        gat(j1, rows1, sem1)
        pltpu.make_async_copy(h2_hbm.at[src_v.at[j0]], rows0, sem0).wait()
        pltpu.sync_copy(rows0, acc.at[dst_v.at[j0]], add=True)

        @pl.when(j2 < NCHUNK)
        def _():
            gat(j2, rows0, sem0)

        pltpu.make_async_copy(h2_hbm.at[src_v.at[j1]], rows1, sem1).wait()
        pltpu.sync_copy(rows1, acc.at[dst_v.at[j1]], add=True)
        return carry

    lax.fori_loop(0, NCHUNK // 2, body, 0)
    plsc.subcore_barrier()

    sl = pl.ds(sid * ROWS_PER_TILE, ROWS_PER_TILE)
    pltpu.sync_copy(acc.at[sl], agg_out.at[cid, sl])


_BLK = 512
_GRID = N_PAD // _BLK


def _h2_body(x_ref, w_ref, deg_ref, h2b_ref):
    deg = jnp.maximum(deg_ref[0, :] + deg_ref[1, :], 1.0)
    dis = lax.rsqrt(deg)
    h = jnp.dot(x_ref[...], w_ref[...], preferred_element_type=jnp.float32)
    h2b_ref[...] = (h * dis[:, None]).astype(jnp.bfloat16)


def _h2_call(x_pad, W, degp):
    return pl.pallas_call(
        _h2_body,
        grid=(_GRID,),
        in_specs=[
            pl.BlockSpec((_BLK, D), lambda i: (i, 0)),
            pl.BlockSpec((D, D), lambda i: (0, 0)),
            pl.BlockSpec((NC, _BLK), lambda i: (0, i)),
        ],
        out_specs=pl.BlockSpec((_BLK, D), lambda i: (i, 0)),
        out_shape=jax.ShapeDtypeStruct((N_PAD, D), jnp.bfloat16),
    )(x_pad, W, degp)


def _out_body(agg_ref, deg_ref, b_ref, out_ref):
    deg = jnp.maximum(deg_ref[:, 0] + deg_ref[:, 1], 1.0)
    dis = lax.rsqrt(deg)
    s = agg_ref[0].astype(jnp.float32) + agg_ref[1].astype(jnp.float32)
    out_ref[...] = jnp.maximum(s * dis[:, None] + b_ref[...], 0.0)


_OBLK = 400
_OGRID = N_NODES // _OBLK


def _out_call(agg, degp_t, b2):
    return pl.pallas_call(
        _out_body,
        grid=(_OGRID,),
        in_specs=[
            pl.BlockSpec((NC, _OBLK, D), lambda i: (0, i, 0)),
            pl.BlockSpec((_OBLK, NC), lambda i: (i, 0)),
            pl.BlockSpec((1, D), lambda i: (0, 0)),
        ],
        out_specs=pl.BlockSpec((_OBLK, D), lambda i: (i, 0)),
        out_shape=jax.ShapeDtypeStruct((N_NODES, D), jnp.float32),
    )(agg, degp_t, b2)


def kernel(x, edge_index, W, b):
    src = edge_index[0].astype(jnp.int32)
    dst = edge_index[1].astype(jnp.int32)

    n_fill = E_PAD - N_EDGES - N_NODES
    loop = jnp.arange(N_NODES, dtype=jnp.int32)
    karr = jnp.arange(n_fill, dtype=jnp.int32)
    ex_src = jnp.concatenate([loop, karr % N_NODES])
    ex_dst = jnp.concatenate([loop, N_NODES + karr % (N_PAD - N_NODES)])

    def layout(real, extra):
        return jnp.concatenate(
            [real.reshape(NW, N_EDGES // NW), extra.reshape(NW, EXTRA_PER_TILE)],
            axis=1).reshape(NW * NCHUNK, CHUNK)

    src_all = layout(src, ex_src)
    dst_all = layout(dst, ex_dst)

    degp = _deg_kernel(dst_all)

    x_pad = jnp.pad(x, ((0, N_PAD - N_NODES), (0, 0)))
    h2b = _h2_call(x_pad, W, degp)

    agg = _agg_kernel(src_all, dst_all, h2b)

    return _out_call(agg, degp.T, b.reshape(1, D))
